# baseline (speedup 1.0000x reference)
import jax
import jax.numpy as jnp
from jax.experimental import pallas as pl
from jax.experimental.pallas import tpu as pltpu

NCHUNK = 8


def _dma_copy_kernel(k_hbm, v_hbm, ko, vo, kbuf, vbuf, rsk, rsv, wsk, wsv):
    b = k_hbm.shape[0]
    c = b // NCHUNK
    k_reads = []
    v_reads = []
    for i in range(NCHUNK):
        sl = pl.ds(i * c, c)
        rk = pltpu.make_async_copy(k_hbm.at[sl], kbuf.at[sl], rsk.at[i])
        rv = pltpu.make_async_copy(v_hbm.at[sl], vbuf.at[sl], rsv.at[i])
        rk.start()
        rv.start()
        k_reads.append(rk)
        v_reads.append(rv)
    writes = []
    for i in range(NCHUNK):
        sl = pl.ds(i * c, c)
        k_reads[i].wait()
        wk = pltpu.make_async_copy(kbuf.at[sl], ko.at[sl], wsk.at[i])
        wk.start()
        writes.append(wk)
        v_reads[i].wait()
        wv = pltpu.make_async_copy(vbuf.at[sl], vo.at[sl], wsv.at[i])
        wv.start()
        writes.append(wv)
    for w in writes:
        w.wait()


def kernel(k_val, v_val, k_cache, v_cache):
    del k_cache, v_cache
    shape = k_val.shape
    k_out, v_out = pl.pallas_call(
        _dma_copy_kernel,
        in_specs=[
            pl.BlockSpec(memory_space=pl.ANY),
            pl.BlockSpec(memory_space=pl.ANY),
        ],
        out_specs=[
            pl.BlockSpec(memory_space=pl.ANY),
            pl.BlockSpec(memory_space=pl.ANY),
        ],
        out_shape=[
            jax.ShapeDtypeStruct(shape, k_val.dtype),
            jax.ShapeDtypeStruct(shape, v_val.dtype),
        ],
        scratch_shapes=[
            pltpu.VMEM(shape, jnp.float32),
            pltpu.VMEM(shape, jnp.float32),
            pltpu.SemaphoreType.DMA((NCHUNK,)),
            pltpu.SemaphoreType.DMA((NCHUNK,)),
            pltpu.SemaphoreType.DMA((NCHUNK,)),
            pltpu.SemaphoreType.DMA((NCHUNK,)),
        ],
    )(k_val, v_val)
    return (k_out, v_out)


# speedup vs baseline: 1.0382x; 1.0382x over previous
import jax
import jax.numpy as jnp
from jax.experimental import pallas as pl


def _copy_kernel(k_ref, v_ref, k_out_ref, v_out_ref):
    k_out_ref[...] = k_ref[...]
    v_out_ref[...] = v_ref[...]


def kernel(k_val, v_val, k_cache, v_cache):
    del k_cache, v_cache
    b, s, h, d = k_val.shape
    blk = 8
    spec = pl.BlockSpec((blk, s, h, d), lambda i: (i, 0, 0, 0))
    k_out, v_out = pl.pallas_call(
        _copy_kernel,
        grid=(b // blk,),
        in_specs=[spec, spec],
        out_specs=[spec, spec],
        out_shape=[
            jax.ShapeDtypeStruct((b, s, h, d), k_val.dtype),
            jax.ShapeDtypeStruct((b, s, h, d), v_val.dtype),
        ],
    )(k_val, v_val)
    return (k_out, v_out)
